# baseline (speedup 1.0000x reference)
import math

import jax
import jax.numpy as jnp
from jax import lax
from jax.experimental import pallas as pl
from jax.experimental.pallas import tpu as pltpu
import jax.experimental.pallas.tpu_sc as plsc

_N = 10000
_E = 320000
_D = 128
_H = 8
_DK = 16
_SCALE = 1.0 / math.sqrt(_DK)

_NC = 2
_NS = 16
_NW = _NC * _NS
_EPW = _E // _NW
_C = 40
_NCHUNK = _EPW // _C
_NPAD = 10240
_ROWS_PER_TILE = _NPAD // _NS
_DROWS = _NPAD // 16
_TROWS = _NPAD + _DROWS
_TROWS_PER_TILE = _TROWS // _NS


def _proj_body(x_ref, wq_ref, bq_ref, wk_ref, bk_ref, wv_ref, bv_ref,
               q_ref, kv_ref):
    xb = x_ref[...]
    q = jnp.dot(xb, wq_ref[...], preferred_element_type=jnp.float32)
    q = (q + bq_ref[...]) * _SCALE
    k = jnp.dot(xb, wk_ref[...], preferred_element_type=jnp.float32) + bk_ref[...]
    v = jnp.dot(xb, wv_ref[...], preferred_element_type=jnp.float32) + bv_ref[...]
    q_ref[...] = q
    kv_ref[...] = jnp.concatenate([k, v], axis=1)


def _proj(x, wq, bq, wk, bk, wv, bv):
    nb = 10
    br = _N // nb
    full = pl.BlockSpec((_D, _D), lambda i: (0, 0))
    bias = pl.BlockSpec((1, _D), lambda i: (0, 0))
    return pl.pallas_call(
        _proj_body,
        grid=(nb,),
        in_specs=[pl.BlockSpec((br, _D), lambda i: (i, 0)),
                  full, bias, full, bias, full, bias],
        out_specs=[pl.BlockSpec((br, _D), lambda i: (i, 0)),
                   pl.BlockSpec((br, 2 * _D), lambda i: (i, 0))],
        out_shape=[jax.ShapeDtypeStruct((_N, _D), jnp.float32),
                   jax.ShapeDtypeStruct((_N, 2 * _D), jnp.float32)],
    )(x, wq, bq, wk, bk, wv, bv)


def _edge_body(q_hbm, kv_hbm, src_hbm, dst_hbm, logits_hbm, acc_hbm,
               src_v, dst_v, ci_v, q_v, kv_v, lg_v, cb_v,
               table, si0, si1, sg0, sg1, slg, sct):
    cid = lax.axis_index("c")
    sid = lax.axis_index("s")
    wid = cid * _NS + sid
    lanes = lax.iota(jnp.int32, 16)
    zeros16 = jnp.zeros((16,), jnp.float32)
    si = (si0, si1)
    sg = (sg0, sg1)

    def _zc(t, _):
        i = t // (_D // 16)
        j = t % (_D // 16)
        cb_v[i, pl.ds(j * 16, 16)] = zeros16
        return 0
    lax.fori_loop(0, 2 * _C * (_D // 16), _zc, 0)
    tbase = sid * _TROWS_PER_TILE
    for r in range(_TROWS_PER_TILE // (2 * _C)):
        pltpu.sync_copy(cb_v, table.at[pl.ds(tbase + r * 2 * _C, 2 * _C)])
    pltpu.sync_copy(
        cb_v.at[pl.ds(0, _TROWS_PER_TILE % (2 * _C))],
        table.at[pl.ds(tbase + (_TROWS_PER_TILE // (2 * _C)) * 2 * _C,
                       _TROWS_PER_TILE % (2 * _C))])
    plsc.subcore_barrier()

    ebase = wid * _EPW
    hcol = (lanes & 7) * _DK
    high8 = lanes >= 8

    def issue_idx(c, p):
        off = ebase + c * _C
        pltpu.async_copy(src_hbm.at[pl.ds(off, _C)], src_v.at[p], si[p])
        pltpu.async_copy(dst_hbm.at[pl.ds(off, _C)], dst_v.at[p], si[p])

    def wait_idx(p):
        pltpu.make_async_copy(
            src_hbm.at[pl.ds(0, _C)], src_v.at[p], si[p]).wait()
        pltpu.make_async_copy(
            dst_hbm.at[pl.ds(0, _C)], dst_v.at[p], si[p]).wait()

    def issue_gather(p):
        pltpu.async_copy(q_hbm.at[src_v.at[p]], q_v.at[p], sg[p])
        pltpu.async_copy(kv_hbm.at[dst_v.at[p]], kv_v.at[p], sg[p])

    def wait_gather(p):
        pltpu.make_async_copy(q_hbm.at[src_v.at[p]], q_v.at[p], sg[p]).wait()
        pltpu.make_async_copy(
            kv_hbm.at[dst_v.at[p]], kv_v.at[p], sg[p]).wait()

    def wait_flush():
        pltpu.make_async_copy(
            lg_v, logits_hbm.at[pl.ds(0, _C * _H)], slg).wait()
        pltpu.make_async_copy(cb_v, table.at[ci_v], sct).wait()

    def compute_and_flush(c, p):
        for t in (0, 16, _C - 16):
            sv = src_v[p, pl.ds(t, 16)]
            ci_v[pl.ds(t, 16)] = sv
            ci_v[pl.ds(_C + t, 16)] = (
                lax.shift_right_logical(sv, 4) + _NPAD)

        def _pair(j, _):
            i0 = j * 2
            row = i0 + jnp.where(high8, 1, 0)
            pvec = jnp.full((16,), p, jnp.int32)
            a = [zeros16] * 4
            for d in range(_DK):
                gq = plsc.load_gather(q_v, [pvec, row, hcol + d])
                gk = plsc.load_gather(kv_v, [pvec, row, hcol + d])
                a[d % 4] = a[d % 4] + gq * gk
            lg = (a[0] + a[1]) + (a[2] + a[3])
            lg_v[pl.ds(j * 16, 16)] = lg
            ev = jnp.exp(lg)
            for h in range(_D // 16):
                cb_v[_C + i0, pl.ds(h * 16, 16)] = zeros16
                cb_v[_C + i0 + 1, pl.ds(h * 16, 16)] = zeros16
            sb = plsc.load_gather(ci_v, [row])
            dcol = (sb & 15) * _H + (lanes & 7)
            plsc.store_scatter(cb_v, [_C + row, dcol], ev)
            for d in range(_DK):
                gv = plsc.load_gather(kv_v, [pvec, row, _D + hcol + d])
                plsc.store_scatter(cb_v, [row, hcol + d], ev * gv)
            return 0
        lax.fori_loop(0, _C // 2, _pair, 0)

        off = ebase + c * _C
        pltpu.async_copy(lg_v, logits_hbm.at[pl.ds(off * _H, _C * _H)], slg)
        pltpu.async_copy(cb_v, table.at[ci_v], sct, add=True)

    issue_idx(0, 0)
    issue_idx(1, 1)
    wait_idx(0)
    issue_gather(0)

    def _super(g, _):
        c0 = g * 2
        wait_idx(1)
        issue_gather(1)
        @pl.when(g > 0)
        def _():
            wait_flush()
        wait_gather(0)
        @pl.when(g < (_NCHUNK // 2) - 1)
        def _():
            issue_idx(c0 + 2, 0)
        compute_and_flush(c0, 0)
        @pl.when(g < (_NCHUNK // 2) - 1)
        def _():
            wait_idx(0)
            issue_gather(0)
        wait_flush()
        wait_gather(1)
        @pl.when(g < (_NCHUNK // 2) - 1)
        def _():
            issue_idx(c0 + 3, 1)
        compute_and_flush(c0 + 1, 1)
        return 0
    lax.fori_loop(0, _NCHUNK // 2, _super, 0)
    wait_flush()

    plsc.subcore_barrier()
    pltpu.sync_copy(
        table.at[pl.ds(tbase, _TROWS_PER_TILE)],
        acc_hbm.at[cid, pl.ds(tbase, _TROWS_PER_TILE)])


def _make_edge():
    return pl.kernel(
        _edge_body,
        out_type=[jax.ShapeDtypeStruct((_E * _H,), jnp.float32),
                  jax.ShapeDtypeStruct((_NC, _TROWS, _D), jnp.float32)],
        mesh=plsc.VectorSubcoreMesh(core_axis_name="c", subcore_axis_name="s"),
        compiler_params=pltpu.CompilerParams(needs_layout_passes=False),
        scratch_types=[
            pltpu.VMEM((2, _C), jnp.int32),
            pltpu.VMEM((2, _C), jnp.int32),
            pltpu.VMEM((2 * _C,), jnp.int32),
            pltpu.VMEM((2, _C, _D), jnp.float32),
            pltpu.VMEM((2, _C, 2 * _D), jnp.float32),
            pltpu.VMEM((_C * _H,), jnp.float32),
            pltpu.VMEM((2 * _C, _D), jnp.float32),
            pltpu.VMEM_SHARED((_TROWS, _D), jnp.float32),
            pltpu.SemaphoreType.DMA,
            pltpu.SemaphoreType.DMA,
            pltpu.SemaphoreType.DMA,
            pltpu.SemaphoreType.DMA,
            pltpu.SemaphoreType.DMA,
            pltpu.SemaphoreType.DMA,
        ],
    )


def _out_body(a0_ref, a1_ref, den_ref, wo_ref, bo_ref, o_ref):
    attn = a0_ref[...] + a1_ref[...]
    den8 = den_ref[...]
    r_i = lax.broadcasted_iota(jnp.int32, (_H, _D), 0)
    c_i = lax.broadcasted_iota(jnp.int32, (_H, _D), 1)
    expand = jnp.where((c_i // _DK) == r_i, 1.0, 0.0)
    den = jnp.dot(den8, expand, preferred_element_type=jnp.float32)
    nz = den != 0.0
    attn = jnp.where(nz, attn / jnp.where(nz, den, 1.0), 0.0)
    o_ref[...] = (jnp.dot(attn, wo_ref[...], preferred_element_type=jnp.float32)
                  + bo_ref[...])


def _outproj(a0, a1, den, wo, bo):
    nb = 10
    br = _NPAD // nb
    return pl.pallas_call(
        _out_body,
        grid=(nb,),
        in_specs=[pl.BlockSpec((br, _D), lambda i: (i, 0)),
                  pl.BlockSpec((br, _D), lambda i: (i, 0)),
                  pl.BlockSpec((br, _H), lambda i: (i, 0)),
                  pl.BlockSpec((_D, _D), lambda i: (0, 0)),
                  pl.BlockSpec((1, _D), lambda i: (0, 0))],
        out_specs=pl.BlockSpec((br, _D), lambda i: (i, 0)),
        out_shape=jax.ShapeDtypeStruct((_NPAD, _D), jnp.float32),
    )(a0, a1, den, wo, bo)


def kernel(x, edge_index, Wq, bq, Wk, bk, Wv, bv, Wo, bo):
    src = edge_index[0]
    dst = edge_index[1]
    q, kv = _proj(x, Wq, bq.reshape(1, _D), Wk, bk.reshape(1, _D),
                  Wv, bv.reshape(1, _D))
    logits_flat, accden = _make_edge()(q, kv, src, dst)
    den = (accden[0, _NPAD:] + accden[1, _NPAD:]).reshape(_NPAD, _H)
    out = _outproj(accden[0, :_NPAD], accden[1, :_NPAD], den, Wo,
                   bo.reshape(1, _D))[:_N]
    logits = logits_flat.reshape(_E, _H, 1)
    return out, logits


# speedup vs baseline: 1.3023x; 1.3023x over previous
import math

import jax
import jax.numpy as jnp
from jax import lax
from jax.experimental import pallas as pl
from jax.experimental.pallas import tpu as pltpu
import jax.experimental.pallas.tpu_sc as plsc

_N = 10000
_E = 320000
_D = 128
_H = 8
_DK = 16
_SCALE = 1.0 / math.sqrt(_DK)

_NC = 2
_NS = 16
_NW = _NC * _NS
_EPW = _E // _NW
_C = 40
_NCHUNK = _EPW // _C
_NPAD = 10240
_ROWS_PER_TILE = _NPAD // _NS
_DROWS = _NPAD // 16
_TROWS = _NPAD + _DROWS
_TROWS_PER_TILE = _TROWS // _NS


def _proj_body(x_ref, wq_ref, bq_ref, wk_ref, bk_ref, wv_ref, bv_ref,
               q_ref, kv_ref):
    xb = x_ref[...]
    q = jnp.dot(xb, wq_ref[...], preferred_element_type=jnp.float32)
    q = (q + bq_ref[...]) * _SCALE
    k = jnp.dot(xb, wk_ref[...], preferred_element_type=jnp.float32) + bk_ref[...]
    v = jnp.dot(xb, wv_ref[...], preferred_element_type=jnp.float32) + bv_ref[...]
    q_ref[...] = q
    kv_ref[...] = jnp.concatenate([k, v], axis=1)


def _proj(x, wq, bq, wk, bk, wv, bv):
    nb = 10
    br = _N // nb
    full = pl.BlockSpec((_D, _D), lambda i: (0, 0))
    bias = pl.BlockSpec((1, _D), lambda i: (0, 0))
    return pl.pallas_call(
        _proj_body,
        grid=(nb,),
        in_specs=[pl.BlockSpec((br, _D), lambda i: (i, 0)),
                  full, bias, full, bias, full, bias],
        out_specs=[pl.BlockSpec((br, _D), lambda i: (i, 0)),
                   pl.BlockSpec((br, 2 * _D), lambda i: (i, 0))],
        out_shape=[jax.ShapeDtypeStruct((_N, _D), jnp.float32),
                   jax.ShapeDtypeStruct((_N, 2 * _D), jnp.float32)],
    )(x, wq, bq, wk, bk, wv, bv)


def _edge_body(q_hbm, kv_hbm, src_hbm, dst_hbm, logits_hbm, acc_hbm,
               src_v, dst_v, ci_v, q_v, kv_v, lg_v, cb_v,
               table, si0, si1, sg0, sg1, slg, sct):
    cid = lax.axis_index("c")
    sid = lax.axis_index("s")
    wid = cid * _NS + sid
    lanes = lax.iota(jnp.int32, 16)
    zeros16 = jnp.zeros((16,), jnp.float32)
    si = (si0, si1)
    sg = (sg0, sg1)

    def _zc(t, _):
        i = t // (_D // 16)
        j = t % (_D // 16)
        cb_v[i, pl.ds(j * 16, 16)] = zeros16
        return 0
    lax.fori_loop(0, 2 * _C * (_D // 16), _zc, 0)
    tbase = sid * _TROWS_PER_TILE
    for r in range(_TROWS_PER_TILE // (2 * _C)):
        pltpu.sync_copy(cb_v, table.at[pl.ds(tbase + r * 2 * _C, 2 * _C)])
    pltpu.sync_copy(
        cb_v.at[pl.ds(0, _TROWS_PER_TILE % (2 * _C))],
        table.at[pl.ds(tbase + (_TROWS_PER_TILE // (2 * _C)) * 2 * _C,
                       _TROWS_PER_TILE % (2 * _C))])
    plsc.subcore_barrier()

    ebase = wid * _EPW
    hcol = (lanes & 7) * _DK
    high8 = lanes >= 8

    def issue_idx(c, p):
        off = ebase + c * _C
        pltpu.async_copy(src_hbm.at[pl.ds(off, _C)], src_v.at[p], si[p])
        pltpu.async_copy(dst_hbm.at[pl.ds(off, _C)], dst_v.at[p], si[p])

    def wait_idx(p):
        pltpu.make_async_copy(
            src_hbm.at[pl.ds(0, _C)], src_v.at[p], si[p]).wait()
        pltpu.make_async_copy(
            dst_hbm.at[pl.ds(0, _C)], dst_v.at[p], si[p]).wait()

    def issue_gather(p):
        pltpu.async_copy(q_hbm.at[src_v.at[p]], q_v.at[p], sg[p])
        pltpu.async_copy(kv_hbm.at[dst_v.at[p]], kv_v.at[p], sg[p])

    def wait_gather(p):
        pltpu.make_async_copy(q_hbm.at[src_v.at[p]], q_v.at[p], sg[p]).wait()
        pltpu.make_async_copy(
            kv_hbm.at[dst_v.at[p]], kv_v.at[p], sg[p]).wait()

    def wait_flush():
        pltpu.make_async_copy(
            lg_v, logits_hbm.at[pl.ds(0, _C * _H)], slg).wait()
        pltpu.make_async_copy(cb_v, table.at[ci_v], sct).wait()

    def compute_and_flush(c, p):
        for t in (0, 16, _C - 16):
            sv = src_v[p, pl.ds(t, 16)]
            ci_v[pl.ds(t, 16)] = sv
            ci_v[pl.ds(_C + t, 16)] = (
                lax.shift_right_logical(sv, 4) + _NPAD)

        def _pair(j, _):
            i0 = j * 2
            row = i0 + jnp.where(high8, 1, 0)
            pvec = jnp.full((16,), p, jnp.int32)
            a = [zeros16] * 4
            for d in range(_DK):
                gq = plsc.load_gather(q_v, [pvec, row, hcol + d])
                gk = plsc.load_gather(kv_v, [pvec, row, hcol + d])
                a[d % 4] = a[d % 4] + gq * gk
            lg = (a[0] + a[1]) + (a[2] + a[3])
            lg_v[pl.ds(j * 16, 16)] = lg
            ev = jnp.exp(lg)
            for h in range(_D // 16):
                cb_v[_C + i0, pl.ds(h * 16, 16)] = zeros16
                cb_v[_C + i0 + 1, pl.ds(h * 16, 16)] = zeros16
            sb = plsc.load_gather(ci_v, [row])
            dcol = (sb & 15) * _H + (lanes & 7)
            plsc.store_scatter(cb_v, [_C + row, dcol], ev)
            for ii in range(2):
                i = i0 + ii
                for h in range(8):
                    eb = jnp.take(ev, jnp.full((16,), ii * 8 + h, jnp.int32),
                                  mode="fill")
                    vv = kv_v[p, i, pl.ds(_D + h * 16, 16)]
                    cb_v[i, pl.ds(h * 16, 16)] = vv * eb
            return 0
        lax.fori_loop(0, _C // 2, _pair, 0)

        off = ebase + c * _C
        pltpu.async_copy(lg_v, logits_hbm.at[pl.ds(off * _H, _C * _H)], slg)
        pltpu.async_copy(cb_v, table.at[ci_v], sct, add=True)

    issue_idx(0, 0)
    issue_idx(1, 1)
    wait_idx(0)
    issue_gather(0)

    def _super(g, _):
        c0 = g * 2
        wait_idx(1)
        issue_gather(1)
        @pl.when(g > 0)
        def _():
            wait_flush()
        wait_gather(0)
        @pl.when(g < (_NCHUNK // 2) - 1)
        def _():
            issue_idx(c0 + 2, 0)
        compute_and_flush(c0, 0)
        @pl.when(g < (_NCHUNK // 2) - 1)
        def _():
            wait_idx(0)
            issue_gather(0)
        wait_flush()
        wait_gather(1)
        @pl.when(g < (_NCHUNK // 2) - 1)
        def _():
            issue_idx(c0 + 3, 1)
        compute_and_flush(c0 + 1, 1)
        return 0
    lax.fori_loop(0, _NCHUNK // 2, _super, 0)
    wait_flush()

    plsc.subcore_barrier()
    pltpu.sync_copy(
        table.at[pl.ds(tbase, _TROWS_PER_TILE)],
        acc_hbm.at[cid, pl.ds(tbase, _TROWS_PER_TILE)])


def _make_edge():
    return pl.kernel(
        _edge_body,
        out_type=[jax.ShapeDtypeStruct((_E * _H,), jnp.float32),
                  jax.ShapeDtypeStruct((_NC, _TROWS, _D), jnp.float32)],
        mesh=plsc.VectorSubcoreMesh(core_axis_name="c", subcore_axis_name="s"),
        compiler_params=pltpu.CompilerParams(needs_layout_passes=False),
        scratch_types=[
            pltpu.VMEM((2, _C), jnp.int32),
            pltpu.VMEM((2, _C), jnp.int32),
            pltpu.VMEM((2 * _C,), jnp.int32),
            pltpu.VMEM((2, _C, _D), jnp.float32),
            pltpu.VMEM((2, _C, 2 * _D), jnp.float32),
            pltpu.VMEM((_C * _H,), jnp.float32),
            pltpu.VMEM((2 * _C, _D), jnp.float32),
            pltpu.VMEM_SHARED((_TROWS, _D), jnp.float32),
            pltpu.SemaphoreType.DMA,
            pltpu.SemaphoreType.DMA,
            pltpu.SemaphoreType.DMA,
            pltpu.SemaphoreType.DMA,
            pltpu.SemaphoreType.DMA,
            pltpu.SemaphoreType.DMA,
        ],
    )


def _out_body(a0_ref, a1_ref, den_ref, wo_ref, bo_ref, o_ref):
    attn = a0_ref[...] + a1_ref[...]
    den8 = den_ref[...]
    r_i = lax.broadcasted_iota(jnp.int32, (_H, _D), 0)
    c_i = lax.broadcasted_iota(jnp.int32, (_H, _D), 1)
    expand = jnp.where((c_i // _DK) == r_i, 1.0, 0.0)
    den = jnp.dot(den8, expand, preferred_element_type=jnp.float32)
    nz = den != 0.0
    attn = jnp.where(nz, attn / jnp.where(nz, den, 1.0), 0.0)
    o_ref[...] = (jnp.dot(attn, wo_ref[...], preferred_element_type=jnp.float32)
                  + bo_ref[...])


def _outproj(a0, a1, den, wo, bo):
    nb = 10
    br = _NPAD // nb
    return pl.pallas_call(
        _out_body,
        grid=(nb,),
        in_specs=[pl.BlockSpec((br, _D), lambda i: (i, 0)),
                  pl.BlockSpec((br, _D), lambda i: (i, 0)),
                  pl.BlockSpec((br, _H), lambda i: (i, 0)),
                  pl.BlockSpec((_D, _D), lambda i: (0, 0)),
                  pl.BlockSpec((1, _D), lambda i: (0, 0))],
        out_specs=pl.BlockSpec((br, _D), lambda i: (i, 0)),
        out_shape=jax.ShapeDtypeStruct((_NPAD, _D), jnp.float32),
    )(a0, a1, den, wo, bo)


def kernel(x, edge_index, Wq, bq, Wk, bk, Wv, bv, Wo, bo):
    src = edge_index[0]
    dst = edge_index[1]
    q, kv = _proj(x, Wq, bq.reshape(1, _D), Wk, bk.reshape(1, _D),
                  Wv, bv.reshape(1, _D))
    logits_flat, accden = _make_edge()(q, kv, src, dst)
    den = (accden[0, _NPAD:] + accden[1, _NPAD:]).reshape(_NPAD, _H)
    out = _outproj(accden[0, :_NPAD], accden[1, :_NPAD], den, Wo,
                   bo.reshape(1, _D))[:_N]
    logits = logits_flat.reshape(_E, _H, 1)
    return out, logits


# speedup vs baseline: 1.8063x; 1.3870x over previous
import math

import jax
import jax.numpy as jnp
from jax import lax
from jax.experimental import pallas as pl
from jax.experimental.pallas import tpu as pltpu
import jax.experimental.pallas.tpu_sc as plsc

_N = 10000
_E = 320000
_D = 128
_H = 8
_DK = 16
_SCALE = 1.0 / math.sqrt(_DK)

_NC = 2
_NS = 16
_NW = _NC * _NS
_EPW = _E // _NW
_C = 40
_NCHUNK = _EPW // _C
_NPAD = 10240
_ROWS_PER_TILE = _NPAD // _NS
_DROWS = _NPAD // 16
_TROWS = _NPAD + _DROWS
_TROWS_PER_TILE = _TROWS // _NS


def _proj_body(x_ref, wq_ref, bq_ref, wk_ref, bk_ref, wv_ref, bv_ref,
               q_ref, kv_ref):
    xb = x_ref[...]
    q = jnp.dot(xb, wq_ref[...], preferred_element_type=jnp.float32)
    q = (q + bq_ref[...]) * _SCALE
    k = jnp.dot(xb, wk_ref[...], preferred_element_type=jnp.float32) + bk_ref[...]
    v = jnp.dot(xb, wv_ref[...], preferred_element_type=jnp.float32) + bv_ref[...]
    q_ref[...] = q
    kv_ref[...] = jnp.concatenate([k, v], axis=1)


def _proj(x, wq, bq, wk, bk, wv, bv):
    nb = 10
    br = _N // nb
    full = pl.BlockSpec((_D, _D), lambda i: (0, 0))
    bias = pl.BlockSpec((1, _D), lambda i: (0, 0))
    return pl.pallas_call(
        _proj_body,
        grid=(nb,),
        in_specs=[pl.BlockSpec((br, _D), lambda i: (i, 0)),
                  full, bias, full, bias, full, bias],
        out_specs=[pl.BlockSpec((br, _D), lambda i: (i, 0)),
                   pl.BlockSpec((br, 2 * _D), lambda i: (i, 0))],
        out_shape=[jax.ShapeDtypeStruct((_N, _D), jnp.float32),
                   jax.ShapeDtypeStruct((_N, 2 * _D), jnp.float32)],
    )(x, wq, bq, wk, bk, wv, bv)


def _edge_body(q_hbm, kv_hbm, src_hbm, dst_hbm, logits_hbm, acc_hbm,
               src_v, dst_v, ci_v, q_v, kv_v, lg_v, cb_v,
               table, si0, si1, sg0, sg1, slg, sct):
    cid = lax.axis_index("c")
    sid = lax.axis_index("s")
    wid = cid * _NS + sid
    lanes = lax.iota(jnp.int32, 16)
    zeros16 = jnp.zeros((16,), jnp.float32)
    si = (si0, si1)
    sg = (sg0, sg1)

    def _zc(t, _):
        i = t // (_D // 16)
        j = t % (_D // 16)
        cb_v[i, pl.ds(j * 16, 16)] = zeros16
        return 0
    lax.fori_loop(0, 2 * _C * (_D // 16), _zc, 0)
    tbase = sid * _TROWS_PER_TILE
    for r in range(_TROWS_PER_TILE // (2 * _C)):
        pltpu.sync_copy(cb_v, table.at[pl.ds(tbase + r * 2 * _C, 2 * _C)])
    pltpu.sync_copy(
        cb_v.at[pl.ds(0, _TROWS_PER_TILE % (2 * _C))],
        table.at[pl.ds(tbase + (_TROWS_PER_TILE // (2 * _C)) * 2 * _C,
                       _TROWS_PER_TILE % (2 * _C))])
    plsc.subcore_barrier()

    ebase = wid * _EPW
    hcol = (lanes & 7) * _DK
    high8 = lanes >= 8

    def issue_idx(c, p):
        off = ebase + c * _C
        pltpu.async_copy(src_hbm.at[pl.ds(off, _C)], src_v.at[p], si[p])
        pltpu.async_copy(dst_hbm.at[pl.ds(off, _C)], dst_v.at[p], si[p])

    def wait_idx(p):
        pltpu.make_async_copy(
            src_hbm.at[pl.ds(0, _C)], src_v.at[p], si[p]).wait()
        pltpu.make_async_copy(
            dst_hbm.at[pl.ds(0, _C)], dst_v.at[p], si[p]).wait()

    def issue_gather(p):
        pltpu.async_copy(q_hbm.at[src_v.at[p]], q_v.at[p], sg[p])
        pltpu.async_copy(kv_hbm.at[dst_v.at[p]], kv_v.at[p], sg[p])

    def wait_gather(p):
        pltpu.make_async_copy(q_hbm.at[src_v.at[p]], q_v.at[p], sg[p]).wait()
        pltpu.make_async_copy(
            kv_hbm.at[dst_v.at[p]], kv_v.at[p], sg[p]).wait()

    def wait_flush():
        pltpu.make_async_copy(
            lg_v, logits_hbm.at[pl.ds(0, _C * _H)], slg).wait()
        pltpu.make_async_copy(cb_v, table.at[ci_v], sct).wait()

    def compute_and_flush(c, p):
        for t in (0, 16, _C - 16):
            sv = src_v[p, pl.ds(t, 16)]
            ci_v[pl.ds(t, 16)] = sv
            ci_v[pl.ds(_C + t, 16)] = (
                lax.shift_right_logical(sv, 4) + _NPAD)

        @plsc.parallel_loop(0, _C // 2, unroll=1)
        def _pair(j):
            i0 = j * 2
            row = i0 + jnp.where(high8, 1, 0)
            pvec = jnp.full((16,), p, jnp.int32)
            a = [zeros16] * 4
            for d in range(_DK):
                gq = plsc.load_gather(q_v, [pvec, row, hcol + d])
                gk = plsc.load_gather(kv_v, [pvec, row, hcol + d])
                a[d % 4] = a[d % 4] + gq * gk
            lg = (a[0] + a[1]) + (a[2] + a[3])
            lg_v[pl.ds(j * 16, 16)] = lg
            ev = jnp.exp(lg)
            for h in range(_D // 16):
                cb_v[_C + i0, pl.ds(h * 16, 16)] = zeros16
                cb_v[_C + i0 + 1, pl.ds(h * 16, 16)] = zeros16
            sb = plsc.load_gather(ci_v, [row])
            dcol = (sb & 15) * _H + (lanes & 7)
            plsc.store_scatter(cb_v, [_C + row, dcol], ev)
            for ii in range(2):
                i = i0 + ii
                for h in range(8):
                    eb = jnp.take(ev, jnp.full((16,), ii * 8 + h, jnp.int32),
                                  mode="fill")
                    vv = kv_v[p, i, pl.ds(_D + h * 16, 16)]
                    cb_v[i, pl.ds(h * 16, 16)] = vv * eb

        off = ebase + c * _C
        pltpu.async_copy(lg_v, logits_hbm.at[pl.ds(off * _H, _C * _H)], slg)
        pltpu.async_copy(cb_v, table.at[ci_v], sct, add=True)

    issue_idx(0, 0)
    issue_idx(1, 1)
    wait_idx(0)
    issue_gather(0)

    def _super(g, _):
        c0 = g * 2
        wait_idx(1)
        issue_gather(1)
        @pl.when(g > 0)
        def _():
            wait_flush()
        wait_gather(0)
        @pl.when(g < (_NCHUNK // 2) - 1)
        def _():
            issue_idx(c0 + 2, 0)
        compute_and_flush(c0, 0)
        @pl.when(g < (_NCHUNK // 2) - 1)
        def _():
            wait_idx(0)
            issue_gather(0)
        wait_flush()
        wait_gather(1)
        @pl.when(g < (_NCHUNK // 2) - 1)
        def _():
            issue_idx(c0 + 3, 1)
        compute_and_flush(c0 + 1, 1)
        return 0
    lax.fori_loop(0, _NCHUNK // 2, _super, 0)
    wait_flush()

    plsc.subcore_barrier()
    pltpu.sync_copy(
        table.at[pl.ds(tbase, _TROWS_PER_TILE)],
        acc_hbm.at[cid, pl.ds(tbase, _TROWS_PER_TILE)])


def _make_edge():
    return pl.kernel(
        _edge_body,
        out_type=[jax.ShapeDtypeStruct((_E * _H,), jnp.float32),
                  jax.ShapeDtypeStruct((_NC, _TROWS, _D), jnp.float32)],
        mesh=plsc.VectorSubcoreMesh(core_axis_name="c", subcore_axis_name="s"),
        compiler_params=pltpu.CompilerParams(needs_layout_passes=False),
        scratch_types=[
            pltpu.VMEM((2, _C), jnp.int32),
            pltpu.VMEM((2, _C), jnp.int32),
            pltpu.VMEM((2 * _C,), jnp.int32),
            pltpu.VMEM((2, _C, _D), jnp.float32),
            pltpu.VMEM((2, _C, 2 * _D), jnp.float32),
            pltpu.VMEM((_C * _H,), jnp.float32),
            pltpu.VMEM((2 * _C, _D), jnp.float32),
            pltpu.VMEM_SHARED((_TROWS, _D), jnp.float32),
            pltpu.SemaphoreType.DMA,
            pltpu.SemaphoreType.DMA,
            pltpu.SemaphoreType.DMA,
            pltpu.SemaphoreType.DMA,
            pltpu.SemaphoreType.DMA,
            pltpu.SemaphoreType.DMA,
        ],
    )


def _out_body(a0_ref, a1_ref, den_ref, wo_ref, bo_ref, o_ref):
    attn = a0_ref[...] + a1_ref[...]
    den8 = den_ref[...]
    r_i = lax.broadcasted_iota(jnp.int32, (_H, _D), 0)
    c_i = lax.broadcasted_iota(jnp.int32, (_H, _D), 1)
    expand = jnp.where((c_i // _DK) == r_i, 1.0, 0.0)
    den = jnp.dot(den8, expand, preferred_element_type=jnp.float32)
    nz = den != 0.0
    attn = jnp.where(nz, attn / jnp.where(nz, den, 1.0), 0.0)
    o_ref[...] = (jnp.dot(attn, wo_ref[...], preferred_element_type=jnp.float32)
                  + bo_ref[...])


def _outproj(a0, a1, den, wo, bo):
    nb = 10
    br = _NPAD // nb
    return pl.pallas_call(
        _out_body,
        grid=(nb,),
        in_specs=[pl.BlockSpec((br, _D), lambda i: (i, 0)),
                  pl.BlockSpec((br, _D), lambda i: (i, 0)),
                  pl.BlockSpec((br, _H), lambda i: (i, 0)),
                  pl.BlockSpec((_D, _D), lambda i: (0, 0)),
                  pl.BlockSpec((1, _D), lambda i: (0, 0))],
        out_specs=pl.BlockSpec((br, _D), lambda i: (i, 0)),
        out_shape=jax.ShapeDtypeStruct((_NPAD, _D), jnp.float32),
    )(a0, a1, den, wo, bo)


def kernel(x, edge_index, Wq, bq, Wk, bk, Wv, bv, Wo, bo):
    src = edge_index[0]
    dst = edge_index[1]
    q, kv = _proj(x, Wq, bq.reshape(1, _D), Wk, bk.reshape(1, _D),
                  Wv, bv.reshape(1, _D))
    logits_flat, accden = _make_edge()(q, kv, src, dst)
    den = (accden[0, _NPAD:] + accden[1, _NPAD:]).reshape(_NPAD, _H)
    out = _outproj(accden[0, :_NPAD], accden[1, :_NPAD], den, Wo,
                   bo.reshape(1, _D))[:_N]
    logits = logits_flat.reshape(_E, _H, 1)
    return out, logits


# speedup vs baseline: 1.8293x; 1.0127x over previous
import math

import jax
import jax.numpy as jnp
from jax import lax
from jax.experimental import pallas as pl
from jax.experimental.pallas import tpu as pltpu
import jax.experimental.pallas.tpu_sc as plsc

_N = 10000
_E = 320000
_D = 128
_H = 8
_DK = 16
_SCALE = 1.0 / math.sqrt(_DK)

_NC = 2
_NS = 16
_NW = _NC * _NS
_EPW = _E // _NW
_C = 40
_NCHUNK = _EPW // _C
_NPAD = 10240
_ROWS_PER_TILE = _NPAD // _NS
_DROWS = _NPAD // 16
_TROWS = _NPAD + _DROWS
_TROWS_PER_TILE = _TROWS // _NS


def _proj_body(x_ref, wq_ref, bq_ref, wk_ref, bk_ref, wv_ref, bv_ref,
               q_ref, kv_ref):
    xb = x_ref[...]
    q = jnp.dot(xb, wq_ref[...], preferred_element_type=jnp.float32)
    q = (q + bq_ref[...]) * _SCALE
    k = jnp.dot(xb, wk_ref[...], preferred_element_type=jnp.float32) + bk_ref[...]
    v = jnp.dot(xb, wv_ref[...], preferred_element_type=jnp.float32) + bv_ref[...]
    q_ref[...] = q
    kv_ref[...] = jnp.concatenate([k, v], axis=1)


def _proj(x, wq, bq, wk, bk, wv, bv):
    nb = 10
    br = _N // nb
    full = pl.BlockSpec((_D, _D), lambda i: (0, 0))
    bias = pl.BlockSpec((1, _D), lambda i: (0, 0))
    return pl.pallas_call(
        _proj_body,
        grid=(nb,),
        in_specs=[pl.BlockSpec((br, _D), lambda i: (i, 0)),
                  full, bias, full, bias, full, bias],
        out_specs=[pl.BlockSpec((br, _D), lambda i: (i, 0)),
                   pl.BlockSpec((br, 2 * _D), lambda i: (i, 0))],
        out_shape=[jax.ShapeDtypeStruct((_N, _D), jnp.float32),
                   jax.ShapeDtypeStruct((_N, 2 * _D), jnp.float32)],
    )(x, wq, bq, wk, bk, wv, bv)


def _edge_body(q_hbm, kv_hbm, src_hbm, dst_hbm, logits_hbm, acc_hbm,
               ei_v, gsrc_v, gdst_v, ci_v, q_v, kv_v, lg_v, cb_v,
               table, sei, sg0, sg1, slg, sct):
    cid = lax.axis_index("c")
    sid = lax.axis_index("s")
    wid = cid * _NS + sid
    lanes = lax.iota(jnp.int32, 16)
    zeros16 = jnp.zeros((16,), jnp.float32)
    sg = (sg0, sg1)

    def _zc(t, _):
        i = t // (_D // 16)
        j = t % (_D // 16)
        cb_v[i, pl.ds(j * 16, 16)] = zeros16
        return 0
    lax.fori_loop(0, 2 * _C * (_D // 16), _zc, 0)
    tbase = sid * _TROWS_PER_TILE
    for r in range(_TROWS_PER_TILE // (2 * _C)):
        pltpu.sync_copy(cb_v, table.at[pl.ds(tbase + r * 2 * _C, 2 * _C)])
    pltpu.sync_copy(
        cb_v.at[pl.ds(0, _TROWS_PER_TILE % (2 * _C))],
        table.at[pl.ds(tbase + (_TROWS_PER_TILE // (2 * _C)) * 2 * _C,
                       _TROWS_PER_TILE % (2 * _C))])
    plsc.subcore_barrier()

    ebase = wid * _EPW
    hcol = (lanes & 7) * _DK
    high8 = lanes >= 8
    nsuper = _NCHUNK // 2

    def issue_super_idx(g):
        off = ebase + g * 2 * _C
        pltpu.async_copy(src_hbm.at[pl.ds(off, 2 * _C)], ei_v.at[0], sei)
        pltpu.async_copy(dst_hbm.at[pl.ds(off, 2 * _C)], ei_v.at[1], sei)

    def wait_super_idx():
        pltpu.make_async_copy(
            src_hbm.at[pl.ds(0, 2 * _C)], ei_v.at[0], sei).wait()
        pltpu.make_async_copy(
            dst_hbm.at[pl.ds(0, 2 * _C)], ei_v.at[1], sei).wait()

    def stage_gather_idx():
        for half in (0, 1):
            for t in (0, 16, _C - 16):
                gsrc_v[half, pl.ds(t, 16)] = ei_v[0, pl.ds(half * _C + t, 16)]
                gdst_v[half, pl.ds(t, 16)] = ei_v[1, pl.ds(half * _C + t, 16)]

    def issue_gather(half):
        pltpu.async_copy(q_hbm.at[gsrc_v.at[half]], q_v.at[half], sg[half])
        pltpu.async_copy(kv_hbm.at[gdst_v.at[half]], kv_v.at[half], sg[half])

    def wait_gather(p):
        pltpu.make_async_copy(
            q_hbm.at[gsrc_v.at[p]], q_v.at[p], sg[p]).wait()
        pltpu.make_async_copy(
            kv_hbm.at[gdst_v.at[p]], kv_v.at[p], sg[p]).wait()

    def wait_cb_flush():
        pltpu.make_async_copy(cb_v, table.at[ci_v], sct).wait()

    def wait_lg_flush():
        pltpu.make_async_copy(
            lg_v, logits_hbm.at[pl.ds(0, 2 * _C * _H)], slg).wait()

    def compute_and_flush(p, sp):
        for t in (0, 16, _C - 16):
            sv = gsrc_v[p, pl.ds(t, 16)]
            ci_v[pl.ds(t, 16)] = sv
            ci_v[pl.ds(_C + t, 16)] = (
                lax.shift_right_logical(sv, 4) + _NPAD)

        @plsc.parallel_loop(0, _C // 2, unroll=1)
        def _pair(j):
            i0 = j * 2
            row = i0 + jnp.where(high8, 1, 0)
            pvec = jnp.full((16,), p, jnp.int32)
            a = [zeros16] * 4
            for d in range(_DK):
                gq = plsc.load_gather(q_v, [pvec, row, hcol + d])
                gk = plsc.load_gather(kv_v, [pvec, row, hcol + d])
                a[d % 4] = a[d % 4] + gq * gk
            lg = (a[0] + a[1]) + (a[2] + a[3])
            lg_v[pl.ds(p * _C * _H + j * 16, 16)] = lg
            ev = jnp.exp(lg)
            for h in range(_D // 16):
                cb_v[_C + i0, pl.ds(h * 16, 16)] = zeros16
                cb_v[_C + i0 + 1, pl.ds(h * 16, 16)] = zeros16
            sb = plsc.load_gather(ci_v, [row])
            dcol = (sb & 15) * _H + (lanes & 7)
            plsc.store_scatter(cb_v, [_C + row, dcol], ev)
            for ii in range(2):
                i = i0 + ii
                for h in range(8):
                    eb = jnp.take(ev, jnp.full((16,), ii * 8 + h, jnp.int32),
                                  mode="fill")
                    vv = kv_v[p, i, pl.ds(_D + h * 16, 16)]
                    cb_v[i, pl.ds(h * 16, 16)] = vv * eb

        pltpu.async_copy(cb_v, table.at[ci_v], sct, add=True)

    def flush_lg(g):
        off = ebase + g * 2 * _C
        pltpu.async_copy(
            lg_v, logits_hbm.at[pl.ds(off * _H, 2 * _C * _H)], slg)

    issue_super_idx(0)
    wait_super_idx()
    stage_gather_idx()
    issue_gather(0)

    def _super(g, _):
        sp = g & 1
        issue_gather(1)
        @pl.when(g > 0)
        def _():
            wait_lg_flush()
            wait_cb_flush()
        wait_gather(0)
        @pl.when(g < nsuper - 1)
        def _():
            issue_super_idx(g + 1)
        compute_and_flush(0, sp)
        wait_cb_flush()
        wait_gather(1)
        @pl.when(g < nsuper - 1)
        def _():
            wait_super_idx()
            stage_gather_idx()
            issue_gather(0)
        compute_and_flush(1, sp)
        flush_lg(g)
        return 0
    lax.fori_loop(0, nsuper, _super, 0)
    wait_cb_flush()
    wait_lg_flush()

    plsc.subcore_barrier()
    pltpu.sync_copy(
        table.at[pl.ds(tbase, _TROWS_PER_TILE)],
        acc_hbm.at[cid, pl.ds(tbase, _TROWS_PER_TILE)])


def _make_edge():
    return pl.kernel(
        _edge_body,
        out_type=[jax.ShapeDtypeStruct((_E * _H,), jnp.float32),
                  jax.ShapeDtypeStruct((_NC, _TROWS, _D), jnp.float32)],
        mesh=plsc.VectorSubcoreMesh(core_axis_name="c", subcore_axis_name="s"),
        compiler_params=pltpu.CompilerParams(needs_layout_passes=False),
        scratch_types=[
            pltpu.VMEM((2, 2 * _C), jnp.int32),
            pltpu.VMEM((2, _C), jnp.int32),
            pltpu.VMEM((2, _C), jnp.int32),
            pltpu.VMEM((2 * _C,), jnp.int32),
            pltpu.VMEM((2, _C, _D), jnp.float32),
            pltpu.VMEM((2, _C, 2 * _D), jnp.float32),
            pltpu.VMEM((2 * _C * _H,), jnp.float32),
            pltpu.VMEM((2 * _C, _D), jnp.float32),
            pltpu.VMEM_SHARED((_TROWS, _D), jnp.float32),
            pltpu.SemaphoreType.DMA,
            pltpu.SemaphoreType.DMA,
            pltpu.SemaphoreType.DMA,
            pltpu.SemaphoreType.DMA,
            pltpu.SemaphoreType.DMA,
        ],
    )


def _out_body(a0_ref, a1_ref, den_ref, wo_ref, bo_ref, o_ref):
    attn = a0_ref[...] + a1_ref[...]
    den8 = den_ref[...]
    r_i = lax.broadcasted_iota(jnp.int32, (_H, _D), 0)
    c_i = lax.broadcasted_iota(jnp.int32, (_H, _D), 1)
    expand = jnp.where((c_i // _DK) == r_i, 1.0, 0.0)
    den = jnp.dot(den8, expand, preferred_element_type=jnp.float32)
    nz = den != 0.0
    attn = jnp.where(nz, attn / jnp.where(nz, den, 1.0), 0.0)
    o_ref[...] = (jnp.dot(attn, wo_ref[...], preferred_element_type=jnp.float32)
                  + bo_ref[...])


def _outproj(a0, a1, den, wo, bo):
    nb = 10
    br = _NPAD // nb
    return pl.pallas_call(
        _out_body,
        grid=(nb,),
        in_specs=[pl.BlockSpec((br, _D), lambda i: (i, 0)),
                  pl.BlockSpec((br, _D), lambda i: (i, 0)),
                  pl.BlockSpec((br, _H), lambda i: (i, 0)),
                  pl.BlockSpec((_D, _D), lambda i: (0, 0)),
                  pl.BlockSpec((1, _D), lambda i: (0, 0))],
        out_specs=pl.BlockSpec((br, _D), lambda i: (i, 0)),
        out_shape=jax.ShapeDtypeStruct((_NPAD, _D), jnp.float32),
    )(a0, a1, den, wo, bo)


def kernel(x, edge_index, Wq, bq, Wk, bk, Wv, bv, Wo, bo):
    q, kv = _proj(x, Wq, bq.reshape(1, _D), Wk, bk.reshape(1, _D),
                  Wv, bv.reshape(1, _D))
    logits_flat, accden = _make_edge()(q, kv, edge_index[0], edge_index[1])
    den = (accden[0, _NPAD:] + accden[1, _NPAD:]).reshape(_NPAD, _H)
    out = _outproj(accden[0, :_NPAD], accden[1, :_NPAD], den, Wo,
                   bo.reshape(1, _D))[:_N]
    logits = logits_flat.reshape(_E, _H, 1)
    return out, logits
